# Initial kernel scaffold; baseline (speedup 1.0000x reference)
import jax
import jax.numpy as jnp
from jax.experimental import pallas as pl


def kernel(s, v, z, batch, W, b):
    raise NotImplementedError("write your pallas kernel here")


# speedup vs baseline: 1.3422x; 1.3422x over previous
import functools

import jax
import jax.numpy as jnp
from jax.experimental import pallas as pl
from jax.experimental.pallas import tpu as pltpu

N = 16384
B = 16
SDIM = 256
VDIM3 = 3 * 256
EPS = 1e-06

TILE = 1024
NT = N // TILE


def _stats_kernel(s_ref, v_ref, batch_ref, z_ref, W_ref, b_ref,
                  seg_ref, wb_ref):
    j = pl.program_id(0)

    @pl.when(j == 0)
    def _():
        wb_ref[...] = jax.lax.dot_general(
            z_ref[...], W_ref[...],
            (((1,), (1,)), ((), ())),
            preferred_element_type=jnp.float32,
            precision=jax.lax.Precision.HIGHEST) + b_ref[...]

    s = s_ref[...]
    v = v_ref[...]
    rowsum = jnp.sum(s, axis=1)
    rowsq = jnp.sum(s * s, axis=1)
    vsq = jnp.sum(v * v, axis=1)
    ones = jnp.ones((TILE,), jnp.float32)
    zeros = jnp.zeros((TILE,), jnp.float32)
    stat = jnp.stack([rowsum, rowsq, vsq, ones,
                      zeros, zeros, zeros, zeros], axis=1)

    ids = batch_ref[...]
    onehot = (ids == jax.lax.broadcasted_iota(jnp.int32, (1, B), 1)
              ).astype(jnp.float32)
    partial = jax.lax.dot_general(
        onehot, stat, (((0,), (0,)), ((), ())),
        preferred_element_type=jnp.float32,
        precision=jax.lax.Precision.HIGHEST)

    @pl.when(j == 0)
    def _():
        seg_ref[...] = partial

    @pl.when(j > 0)
    def _():
        seg_ref[...] += partial


def _norm_kernel(s_ref, v_ref, batch_ref, seg_ref, wb_ref,
                 sout_ref, vout_ref):
    seg = seg_ref[...]
    cnt = jnp.clip(seg[:, 3:4], 1.0, None)
    denom = cnt * SDIM
    m = seg[:, 0:1] / denom
    q = seg[:, 1:2] / denom
    var = jnp.clip(q - m * m, EPS, None)
    vm = jnp.clip(seg[:, 2:3] / denom, EPS, None)
    scal = jnp.concatenate([m, 1.0 / var, 1.0 / vm], axis=1)

    ids = batch_ref[...]
    onehot = (ids == jax.lax.broadcasted_iota(jnp.int32, (1, B), 1)
              ).astype(jnp.float32)
    row_wb = jnp.dot(onehot, wb_ref[...],
                     preferred_element_type=jnp.float32,
                     precision=jax.lax.Precision.HIGHEST)
    row_scal = jnp.dot(onehot, scal,
                       preferred_element_type=jnp.float32,
                       precision=jax.lax.Precision.HIGHEST)
    rm = row_scal[:, 0:1]
    riv = row_scal[:, 1:2]
    rivm = row_scal[:, 2:3]

    s = s_ref[...]
    sout_ref[...] = ((s - rm) * riv) * row_wb[:, :SDIM] + row_wb[:, SDIM:]
    vout_ref[...] = v_ref[...] * rivm


@functools.partial(jax.jit, static_argnames=())
def kernel(s, v, z, batch, W, b):
    v2 = v.reshape(N, VDIM3)
    ids = batch.astype(jnp.int32).reshape(N, 1)
    b2 = b.reshape(1, 2 * SDIM)

    seg, wb = pl.pallas_call(
        _stats_kernel,
        grid=(NT,),
        in_specs=[
            pl.BlockSpec((TILE, SDIM), lambda j: (j, 0)),
            pl.BlockSpec((TILE, VDIM3), lambda j: (j, 0)),
            pl.BlockSpec((TILE, 1), lambda j: (j, 0)),
            pl.BlockSpec((B, 256), lambda j: (0, 0)),
            pl.BlockSpec((2 * SDIM, 256), lambda j: (0, 0)),
            pl.BlockSpec((1, 2 * SDIM), lambda j: (0, 0)),
        ],
        out_specs=[
            pl.BlockSpec((B, 8), lambda j: (0, 0)),
            pl.BlockSpec((B, 2 * SDIM), lambda j: (0, 0)),
        ],
        out_shape=[
            jax.ShapeDtypeStruct((B, 8), jnp.float32),
            jax.ShapeDtypeStruct((B, 2 * SDIM), jnp.float32),
        ],
        compiler_params=pltpu.CompilerParams(
            dimension_semantics=("arbitrary",)),
    )(s, v2, ids, z, W, b2)

    sout, vout2 = pl.pallas_call(
        _norm_kernel,
        grid=(NT,),
        in_specs=[
            pl.BlockSpec((TILE, SDIM), lambda j: (j, 0)),
            pl.BlockSpec((TILE, VDIM3), lambda j: (j, 0)),
            pl.BlockSpec((TILE, 1), lambda j: (j, 0)),
            pl.BlockSpec((B, 8), lambda j: (0, 0)),
            pl.BlockSpec((B, 2 * SDIM), lambda j: (0, 0)),
        ],
        out_specs=[
            pl.BlockSpec((TILE, SDIM), lambda j: (j, 0)),
            pl.BlockSpec((TILE, VDIM3), lambda j: (j, 0)),
        ],
        out_shape=[
            jax.ShapeDtypeStruct((N, SDIM), jnp.float32),
            jax.ShapeDtypeStruct((N, VDIM3), jnp.float32),
        ],
        compiler_params=pltpu.CompilerParams(
            dimension_semantics=("arbitrary",)),
    )(s, v2, ids, seg, wb)

    return (sout, vout2.reshape(N, 3, VDIM3 // 3))


# speedup vs baseline: 1.6019x; 1.1935x over previous
import functools

import jax
import jax.numpy as jnp
from jax.experimental import pallas as pl
from jax.experimental.pallas import tpu as pltpu

N = 16384
B = 16
SDIM = 256
VDIM3 = 3 * 256
EPS = 1e-06

TILE = 1024
NT = N // TILE


def _stats_kernel(s_ref, v_ref, batch_ref, z_ref, W_ref, b_ref,
                  seg_ref, wb_ref):
    j = pl.program_id(0)

    @pl.when(j == 0)
    def _():
        wb_ref[...] = jax.lax.dot_general(
            z_ref[...], W_ref[...],
            (((1,), (1,)), ((), ())),
            preferred_element_type=jnp.float32,
            precision=jax.lax.Precision.HIGHEST) + b_ref[...]

    s = s_ref[...]
    rowsum = jnp.sum(s, axis=1)
    rowsq = jnp.sum(s * s, axis=1)
    vsq = jnp.zeros((TILE,), jnp.float32)
    for k in range(3):
        vk = v_ref[:, k, :]
        vsq = vsq + jnp.sum(vk * vk, axis=1)
    ones = jnp.ones((TILE,), jnp.float32)
    zeros = jnp.zeros((TILE,), jnp.float32)
    stat = jnp.stack([rowsum, rowsq, vsq, ones,
                      zeros, zeros, zeros, zeros], axis=1)

    ids = batch_ref[...]
    onehot = (ids == jax.lax.broadcasted_iota(jnp.int32, (1, B), 1)
              ).astype(jnp.float32)
    partial = jax.lax.dot_general(
        onehot, stat, (((0,), (0,)), ((), ())),
        preferred_element_type=jnp.float32,
        precision=jax.lax.Precision.HIGHEST)

    @pl.when(j == 0)
    def _():
        seg_ref[...] = partial

    @pl.when(j > 0)
    def _():
        seg_ref[...] += partial


def _norm_kernel(s_ref, v_ref, batch_ref, seg_ref, wb_ref,
                 sout_ref, vout_ref):
    seg = seg_ref[...]
    cnt = jnp.clip(seg[:, 3:4], 1.0, None)
    denom = cnt * SDIM
    m = seg[:, 0:1] / denom
    q = seg[:, 1:2] / denom
    var = jnp.clip(q - m * m, EPS, None)
    vm = jnp.clip(seg[:, 2:3] / denom, EPS, None)
    scal = jnp.concatenate([m, 1.0 / var, 1.0 / vm], axis=1)

    ids = batch_ref[...]
    onehot = (ids == jax.lax.broadcasted_iota(jnp.int32, (1, B), 1)
              ).astype(jnp.float32)
    row_wb = jnp.dot(onehot, wb_ref[...],
                     preferred_element_type=jnp.float32,
                     precision=jax.lax.Precision.HIGHEST)
    row_scal = jnp.dot(onehot, scal,
                       preferred_element_type=jnp.float32,
                       precision=jax.lax.Precision.HIGHEST)
    rm = row_scal[:, 0:1]
    riv = row_scal[:, 1:2]
    rivm = row_scal[:, 2:3]

    s = s_ref[...]
    sout_ref[...] = ((s - rm) * riv) * row_wb[:, :SDIM] + row_wb[:, SDIM:]
    for k in range(3):
        vout_ref[:, k, :] = v_ref[:, k, :] * rivm


@functools.partial(jax.jit, static_argnames=())
def kernel(s, v, z, batch, W, b):
    ids = batch.astype(jnp.int32).reshape(N, 1)
    b2 = b.reshape(1, 2 * SDIM)

    seg, wb = pl.pallas_call(
        _stats_kernel,
        grid=(NT,),
        in_specs=[
            pl.BlockSpec((TILE, SDIM), lambda j: (j, 0)),
            pl.BlockSpec((TILE, 3, 256), lambda j: (j, 0, 0)),
            pl.BlockSpec((TILE, 1), lambda j: (j, 0)),
            pl.BlockSpec((B, 256), lambda j: (0, 0)),
            pl.BlockSpec((2 * SDIM, 256), lambda j: (0, 0)),
            pl.BlockSpec((1, 2 * SDIM), lambda j: (0, 0)),
        ],
        out_specs=[
            pl.BlockSpec((B, 8), lambda j: (0, 0)),
            pl.BlockSpec((B, 2 * SDIM), lambda j: (0, 0)),
        ],
        out_shape=[
            jax.ShapeDtypeStruct((B, 8), jnp.float32),
            jax.ShapeDtypeStruct((B, 2 * SDIM), jnp.float32),
        ],
        compiler_params=pltpu.CompilerParams(
            dimension_semantics=("arbitrary",)),
    )(s, v, ids, z, W, b2)

    sout, vout = pl.pallas_call(
        _norm_kernel,
        grid=(NT,),
        in_specs=[
            pl.BlockSpec((TILE, SDIM), lambda j: (j, 0)),
            pl.BlockSpec((TILE, 3, 256), lambda j: (j, 0, 0)),
            pl.BlockSpec((TILE, 1), lambda j: (j, 0)),
            pl.BlockSpec((B, 8), lambda j: (0, 0)),
            pl.BlockSpec((B, 2 * SDIM), lambda j: (0, 0)),
        ],
        out_specs=[
            pl.BlockSpec((TILE, SDIM), lambda j: (j, 0)),
            pl.BlockSpec((TILE, 3, 256), lambda j: (j, 0, 0)),
        ],
        out_shape=[
            jax.ShapeDtypeStruct((N, SDIM), jnp.float32),
            jax.ShapeDtypeStruct((N, 3, 256), jnp.float32),
        ],
        compiler_params=pltpu.CompilerParams(
            dimension_semantics=("arbitrary",)),
    )(s, v, ids, seg, wb)

    return (sout, vout)


# speedup vs baseline: 3.7134x; 2.3181x over previous
import functools

import jax
import jax.numpy as jnp
from jax.experimental import pallas as pl
from jax.experimental.pallas import tpu as pltpu

N = 16384
B = 16
SDIM = 256
EPS = 1e-06

TILE = 1024
NT = N // TILE


def _stats_kernel(s_ref, v_ref, batch_ref, z_ref, W_ref, b_ref,
                  seg_ref, wb_ref):
    j = pl.program_id(0)

    @pl.when(j == 0)
    def _():
        wb_ref[...] = jax.lax.dot_general(
            z_ref[...], W_ref[...],
            (((1,), (1,)), ((), ())),
            preferred_element_type=jnp.float32,
            precision=jax.lax.Precision.HIGHEST) + b_ref[...]

    s = s_ref[...]
    rowsum = jnp.sum(s, axis=1, keepdims=True)
    rowsq = jnp.sum(s * s, axis=1, keepdims=True)
    vsq = jnp.zeros((TILE, 1), jnp.float32)
    for k in range(3):
        vk = v_ref[k]
        vsq = vsq + jnp.sum(vk * vk, axis=1, keepdims=True)
    ones = jnp.ones((TILE, 1), jnp.float32)
    zeros = jnp.zeros((TILE, 4), jnp.float32)
    stat = jnp.concatenate([rowsum, rowsq, vsq, ones, zeros], axis=1)

    ids = batch_ref[...]
    onehot = (ids == jax.lax.broadcasted_iota(jnp.int32, (1, B), 1)
              ).astype(jnp.float32)
    partial = jax.lax.dot_general(
        onehot, stat, (((0,), (0,)), ((), ())),
        preferred_element_type=jnp.float32,
        precision=jax.lax.Precision.HIGHEST)

    @pl.when(j == 0)
    def _():
        seg_ref[...] = partial

    @pl.when(j > 0)
    def _():
        seg_ref[...] += partial


def _norm_kernel(s_ref, v_ref, batch_ref, seg_ref, wb_ref,
                 sout_ref, vout_ref):
    seg = seg_ref[...]
    cnt = jnp.clip(seg[:, 3:4], 1.0, None)
    denom = cnt * SDIM
    m = seg[:, 0:1] / denom
    q = seg[:, 1:2] / denom
    var = jnp.clip(q - m * m, EPS, None)
    vm = jnp.clip(seg[:, 2:3] / denom, EPS, None)
    scal = jnp.concatenate([m, 1.0 / var, 1.0 / vm], axis=1)

    ids = batch_ref[...]
    onehot = (ids == jax.lax.broadcasted_iota(jnp.int32, (1, B), 1)
              ).astype(jnp.float32)
    row_wb = jnp.dot(onehot, wb_ref[...],
                     preferred_element_type=jnp.float32,
                     precision=jax.lax.Precision.HIGHEST)
    row_scal = jnp.dot(onehot, scal,
                       preferred_element_type=jnp.float32,
                       precision=jax.lax.Precision.HIGHEST)
    rm = row_scal[:, 0:1]
    riv = row_scal[:, 1:2]
    rivm = row_scal[:, 2:3]

    s = s_ref[...]
    sout_ref[...] = ((s - rm) * riv) * row_wb[:, :SDIM] + row_wb[:, SDIM:]
    for k in range(3):
        vout_ref[k] = v_ref[k] * rivm


@functools.partial(jax.jit, static_argnames=())
def kernel(s, v, z, batch, W, b):
    vp = jnp.transpose(v, (1, 0, 2))
    ids = batch.astype(jnp.int32).reshape(N, 1)
    b2 = b.reshape(1, 2 * SDIM)

    seg, wb = pl.pallas_call(
        _stats_kernel,
        grid=(NT,),
        in_specs=[
            pl.BlockSpec((TILE, SDIM), lambda j: (j, 0)),
            pl.BlockSpec((3, TILE, 256), lambda j: (0, j, 0)),
            pl.BlockSpec((TILE, 1), lambda j: (j, 0)),
            pl.BlockSpec((B, 256), lambda j: (0, 0)),
            pl.BlockSpec((2 * SDIM, 256), lambda j: (0, 0)),
            pl.BlockSpec((1, 2 * SDIM), lambda j: (0, 0)),
        ],
        out_specs=[
            pl.BlockSpec((B, 8), lambda j: (0, 0)),
            pl.BlockSpec((B, 2 * SDIM), lambda j: (0, 0)),
        ],
        out_shape=[
            jax.ShapeDtypeStruct((B, 8), jnp.float32),
            jax.ShapeDtypeStruct((B, 2 * SDIM), jnp.float32),
        ],
        compiler_params=pltpu.CompilerParams(
            dimension_semantics=("arbitrary",)),
    )(s, vp, ids, z, W, b2)

    sout, vout = pl.pallas_call(
        _norm_kernel,
        grid=(NT,),
        in_specs=[
            pl.BlockSpec((TILE, SDIM), lambda j: (j, 0)),
            pl.BlockSpec((3, TILE, 256), lambda j: (0, j, 0)),
            pl.BlockSpec((TILE, 1), lambda j: (j, 0)),
            pl.BlockSpec((B, 8), lambda j: (0, 0)),
            pl.BlockSpec((B, 2 * SDIM), lambda j: (0, 0)),
        ],
        out_specs=[
            pl.BlockSpec((TILE, SDIM), lambda j: (j, 0)),
            pl.BlockSpec((3, TILE, 256), lambda j: (0, j, 0)),
        ],
        out_shape=[
            jax.ShapeDtypeStruct((N, SDIM), jnp.float32),
            jax.ShapeDtypeStruct((3, N, 256), jnp.float32),
        ],
        compiler_params=pltpu.CompilerParams(
            dimension_semantics=("arbitrary",)),
    )(s, vp, ids, seg, wb)

    return (sout, jnp.transpose(vout, (1, 0, 2)))
